# baseline (speedup 1.0000x reference)
import functools

import jax
import jax.numpy as jnp
from jax import lax
from jax.experimental import pallas as pl
from jax.experimental.pallas import tpu as pltpu
from jax.experimental.pallas import tpu_sc as plsc

B = 32
N = 16384
S = 64
L = 16
NV = N // L
W = S + L
RADIUS_SQ = 1.0

_mesh = plsc.VectorSubcoreMesh(core_axis_name="c", subcore_axis_name="s")


@functools.partial(
    pl.kernel,
    mesh=_mesh,
    compiler_params=pltpu.CompilerParams(needs_layout_passes=False,
                                         use_tc_tiling_on_sc=True),
    out_type=jax.ShapeDtypeStruct((B, 8, S), jnp.int32),
    scratch_types=[
        pltpu.VMEM((3, N), jnp.float32),
        pltpu.VMEM((8, W), jnp.int32),
    ],
)
def _octant_query_sc(pcs_hbm, out_hbm, pts, obuf):
    wid = lax.axis_index("s") * 2 + lax.axis_index("c")

    pltpu.sync_copy(pcs_hbm.at[wid], pts)

    neg1 = jnp.full((L,), -1, jnp.int32)
    for o in range(8):
        for j in range(W // L):
            obuf[o, pl.ds(j * L, L)] = neg1

    lane = lax.iota(jnp.int32, L)
    zero = jnp.int32(0)

    def cond(carry):
        i, ptrs = carry
        mn = jnp.minimum(jnp.minimum(jnp.minimum(ptrs[0], ptrs[1]),
                                     jnp.minimum(ptrs[2], ptrs[3])),
                         jnp.minimum(jnp.minimum(ptrs[4], ptrs[5]),
                                     jnp.minimum(ptrs[6], ptrs[7])))
        return (i < NV) & (mn < S)

    def body(carry):
        i, ptrs = carry
        off = i * L
        x = pts[0, pl.ds(off, L)]
        y = pts[1, pl.ds(off, L)]
        z = pts[2, pl.ds(off, L)]
        oct_id = ((x > 0).astype(jnp.int32) * 4
                  + (y > 0).astype(jnp.int32) * 2
                  + (z > 0).astype(jnp.int32))
        within = (x * x + y * y + z * z) <= RADIUS_SQ
        idx_vec = lane + off
        new_ptrs = []
        for o in range(8):
            m = within & (oct_id == o)
            plsc.store_compressed(obuf.at[o, pl.ds(ptrs[o], L)], idx_vec,
                                  mask=m)
            pop = jnp.sum(m.astype(jnp.int32))
            new_ptrs.append(jnp.minimum(ptrs[o] + pop, S))
        return i + 1, tuple(new_ptrs)

    lax.while_loop(cond, body, (zero, (zero,) * 8))

    for o in range(8):
        pltpu.sync_copy(obuf.at[o, pl.ds(0, S)], out_hbm.at[wid, o])


def kernel(pcs):
    return _octant_query_sc(pcs)


# speedup vs baseline: 1.0375x; 1.0375x over previous
import functools

import jax
import jax.numpy as jnp
from jax import lax
from jax.experimental import pallas as pl
from jax.experimental.pallas import tpu as pltpu
from jax.experimental.pallas import tpu_sc as plsc

B = 32
N = 16384
S = 64
L = 16
NV = N // L
RADIUS_SQ = 1.0

_mesh = plsc.VectorSubcoreMesh(core_axis_name="c", subcore_axis_name="s")


@functools.partial(
    pl.kernel,
    mesh=_mesh,
    compiler_params=pltpu.CompilerParams(needs_layout_passes=False),
    out_type=jax.ShapeDtypeStruct((B, 8, S), jnp.int32),
    scratch_types=[
        pltpu.VMEM((3, N), jnp.float32),
        pltpu.VMEM((8, S), jnp.int32),
        pltpu.VMEM((L,), jnp.int32),
        pltpu.SemaphoreType.DMA,
    ],
)
def _octant_query_sc(pcs_hbm, out_hbm, pts, obuf, counts, dsem):
    wid = lax.axis_index("s") * 2 + lax.axis_index("c")

    copy = pltpu.async_copy(pcs_hbm.at[wid], pts, dsem)

    lane = lax.iota(jnp.int32, L)
    neg1 = jnp.full((L,), -1, jnp.int32)
    for o in range(8):
        for j in range(S // L):
            obuf[o, pl.ds(j * L, L)] = neg1
    counts[pl.ds(0, L)] = jnp.where(lane < 8, 0, S)
    ones = jnp.ones((L,), jnp.int32)

    copy.wait()

    def do_vreg(i):
        off = i * L
        x = pts[0, pl.ds(off, L)]
        y = pts[1, pl.ds(off, L)]
        z = pts[2, pl.ds(off, L)]
        oct_id = ((x > 0).astype(jnp.int32) * 4
                  + (y > 0).astype(jnp.int32) * 2
                  + (z > 0).astype(jnp.int32))
        within = (x * x + y * y + z * z) <= RADIUS_SQ
        low = oct_id < 4
        shamt = (oct_id & 3) << 3
        oh = jnp.left_shift(jnp.int32(1), shamt)
        zero = jnp.zeros((L,), jnp.int32)
        ohl = jnp.where(within & low, oh, zero)
        ohh = jnp.where(within & jnp.logical_not(low), oh, zero)
        cuml = plsc.cumsum(ohl)
        cumh = plsc.cumsum(ohh)
        incl = (jnp.where(low, cuml, cumh) >> shamt) & 255
        cnt = plsc.load_gather(counts, [oct_id])
        slot = cnt + incl - 1
        sel = within & (slot < S)
        slot_c = jnp.where(sel, slot, 0)
        plsc.store_scatter(obuf, [oct_id, slot_c], lane + off, mask=sel)
        plsc.addupdate_scatter(counts, [oct_id], ones, mask=within)

    def cond(carry):
        i, done = carry
        return (i < NV) & jnp.logical_not(done)

    def body(carry):
        i, _ = carry
        do_vreg(i)
        do_vreg(i + 1)
        cv = counts[pl.ds(0, L)]
        return i + 2, jnp.all(cv >= S)

    lax.while_loop(cond, body, (jnp.int32(0), jnp.bool_(False)))

    pltpu.sync_copy(obuf, out_hbm.at[wid])


def kernel(pcs):
    return _octant_query_sc(pcs)


# speedup vs baseline: 1.3328x; 1.2846x over previous
import functools

import jax
import jax.numpy as jnp
from jax import lax
from jax.experimental import pallas as pl
from jax.experimental.pallas import tpu as pltpu
from jax.experimental.pallas import tpu_sc as plsc

B = 32
N = 16384
S = 64
L = 16
NV = N // L
RADIUS_SQ = 1.0

_mesh = plsc.VectorSubcoreMesh(core_axis_name="c", subcore_axis_name="s")


@functools.partial(
    pl.kernel,
    mesh=_mesh,
    compiler_params=pltpu.CompilerParams(needs_layout_passes=False),
    out_type=jax.ShapeDtypeStruct((B, 8, S), jnp.int32),
    scratch_types=[
        pltpu.VMEM((3, N), jnp.float32),
        pltpu.VMEM((8, S), jnp.int32),
        pltpu.VMEM((L,), jnp.int32),
        pltpu.SemaphoreType.DMA,
    ],
)
def _octant_query_sc(pcs_hbm, out_hbm, pts, obuf, counts, dsem):
    wid = lax.axis_index("s") * 2 + lax.axis_index("c")

    copy = pltpu.async_copy(pcs_hbm.at[:, wid], pts, dsem)

    lane = lax.iota(jnp.int32, L)
    neg1 = jnp.full((L,), -1, jnp.int32)
    for o in range(8):
        for j in range(S // L):
            obuf[o, pl.ds(j * L, L)] = neg1
    counts[pl.ds(0, L)] = jnp.where(lane < 8, 0, S)
    ones = jnp.ones((L,), jnp.int32)

    copy.wait()

    def do_vreg(i):
        off = i * L
        x = pts[0, pl.ds(off, L)]
        y = pts[1, pl.ds(off, L)]
        z = pts[2, pl.ds(off, L)]
        oct_id = ((x > 0).astype(jnp.int32) * 4
                  + (y > 0).astype(jnp.int32) * 2
                  + (z > 0).astype(jnp.int32))
        within = (x * x + y * y + z * z) <= RADIUS_SQ
        low = oct_id < 4
        shamt = (oct_id & 3) << 3
        oh = jnp.left_shift(jnp.int32(1), shamt)
        zero = jnp.zeros((L,), jnp.int32)
        ohl = jnp.where(within & low, oh, zero)
        ohh = jnp.where(within & jnp.logical_not(low), oh, zero)
        cuml = plsc.cumsum(ohl)
        cumh = plsc.cumsum(ohh)
        incl = (jnp.where(low, cuml, cumh) >> shamt) & 255
        cnt = plsc.load_gather(counts, [oct_id])
        slot = cnt + incl - 1
        sel = within & (slot < S)
        slot_c = jnp.where(sel, slot, 0)
        plsc.store_scatter(obuf, [oct_id, slot_c], lane + off, mask=sel)
        plsc.addupdate_scatter(counts, [oct_id], ones, mask=within)

    def cond(carry):
        i, done = carry
        return (i < NV) & jnp.logical_not(done)

    def body(carry):
        i, _ = carry
        do_vreg(i)
        do_vreg(i + 1)
        cv = counts[pl.ds(0, L)]
        return i + 2, jnp.all(cv >= S)

    lax.while_loop(cond, body, (jnp.int32(0), jnp.bool_(False)))

    pltpu.sync_copy(obuf, out_hbm.at[wid])


def kernel(pcs):
    return _octant_query_sc(jnp.transpose(pcs, (1, 0, 2)))
